# baseline (speedup 1.0000x reference)
import jax
import jax.numpy as jnp
from jax import lax
from jax.experimental import pallas as pl
from jax.experimental.pallas import tpu as pltpu

N_SLOTS_K = 8
EPS = 1e-12
BB = 4


def _greedy_body(features_ref, out_ref, gram_ref):
    _, n, d = features_ref.shape
    iota_row = lax.broadcasted_iota(jnp.int32, (1, n), 1)
    for b in range(BB):
        fb = features_ref[b]
        sal_row = jnp.sqrt(jnp.sum(fb * fb, axis=1)).reshape(1, n)
        denom_row = jnp.maximum(sal_row, EPS)
        gram_ref[b] = lax.dot_general(
            fb, fb, (((1,), (1,)), ((), ())),
            preferred_element_type=jnp.float32,
        )
        mask_row = jnp.ones((1, n), dtype=jnp.float32)
        for r in range(N_SLOTS_K):
            ms = sal_row * mask_row
            mx = jnp.max(ms)
            idx = jnp.min(jnp.where(ms == mx, iota_row, n)).astype(jnp.int32)
            sel = features_ref[b, pl.ds(idx, 1), :]
            out_ref[b, pl.ds(r, 1), :] = sel
            dots = gram_ref[b, pl.ds(idx, 1), :]
            snorm = jnp.maximum(jnp.sqrt(jnp.sum(sel * sel)), EPS)
            sim = dots / (denom_row * snorm)
            mask_row = mask_row * (1.0 - jnp.clip(sim, 0.0, 1.0))


def kernel(batch_size, features, fallback):
    del batch_size, fallback
    b, n, d = features.shape
    return pl.pallas_call(
        _greedy_body,
        grid=(b // BB,),
        in_specs=[pl.BlockSpec((BB, n, d), lambda i: (i, 0, 0))],
        out_specs=pl.BlockSpec((BB, N_SLOTS_K, d), lambda i: (i, 0, 0)),
        out_shape=jax.ShapeDtypeStruct((b, N_SLOTS_K, d), features.dtype),
        scratch_shapes=[pltpu.VMEM((BB, n, n), jnp.float32)],
    )(features)


# speedup vs baseline: 1.3422x; 1.3422x over previous
import jax
import jax.numpy as jnp
from jax import lax
from jax.experimental import pallas as pl

N_SLOTS_K = 8
EPS = 1e-12
BB = 8


def _greedy_body(features_ref, out_ref):
    _, n, d = features_ref.shape
    iota_row = lax.broadcasted_iota(jnp.int32, (1, n), 1)
    for b in range(BB):
        fb = features_ref[b]
        sal_row = jnp.sqrt(jnp.sum(fb * fb, axis=1)).reshape(1, n)
        denom_row = jnp.maximum(sal_row, EPS)
        mask_row = jnp.ones((1, n), dtype=jnp.float32)
        for r in range(N_SLOTS_K):
            ms = sal_row * mask_row
            mx = jnp.max(ms)
            idx = jnp.min(jnp.where(ms == mx, iota_row, n)).astype(jnp.int32)
            sel = features_ref[b, pl.ds(idx, 1), :]
            out_ref[b, pl.ds(r, 1), :] = sel
            dots = jnp.sum(features_ref[b] * sel, axis=1).reshape(1, n)
            snorm = jnp.maximum(jnp.sqrt(jnp.sum(sel * sel)), EPS)
            sim = dots / (denom_row * snorm)
            mask_row = mask_row * (1.0 - jnp.clip(sim, 0.0, 1.0))


def kernel(batch_size, features, fallback):
    del batch_size, fallback
    b, n, d = features.shape
    return pl.pallas_call(
        _greedy_body,
        grid=(b // BB,),
        in_specs=[pl.BlockSpec((BB, n, d), lambda i: (i, 0, 0))],
        out_specs=pl.BlockSpec((BB, N_SLOTS_K, d), lambda i: (i, 0, 0)),
        out_shape=jax.ShapeDtypeStruct((b, N_SLOTS_K, d), features.dtype),
    )(features)


# speedup vs baseline: 1.3971x; 1.0409x over previous
import jax
import jax.numpy as jnp
from jax import lax
from jax.experimental import pallas as pl

N_SLOTS_K = 8
EPS = 1e-12
BB = 8


LANES = 128


def _chunked_rowdot(a_ref, b_idx, other, n, d):
    acc = a_ref[b_idx, :, pl.ds(0, LANES)] * other[:, 0:LANES]
    for k in range(1, d // LANES):
        acc = acc + a_ref[b_idx, :, pl.ds(k * LANES, LANES)] * other[:, k * LANES:(k + 1) * LANES]
    return jnp.sum(acc, axis=1).reshape(1, n)


def _greedy_body(features_ref, out_ref):
    _, n, d = features_ref.shape
    iota_row = lax.broadcasted_iota(jnp.int32, (1, n), 1)
    for b in range(BB):
        sal2 = features_ref[b, :, pl.ds(0, LANES)] ** 2
        for k in range(1, d // LANES):
            sal2 = sal2 + features_ref[b, :, pl.ds(k * LANES, LANES)] ** 2
        sal_row = jnp.sqrt(jnp.sum(sal2, axis=1)).reshape(1, n)
        denom_row = jnp.maximum(sal_row, EPS)
        mask_row = jnp.ones((1, n), dtype=jnp.float32)
        for r in range(N_SLOTS_K):
            ms = sal_row * mask_row
            mx = jnp.max(ms, axis=1, keepdims=True)
            idx = jnp.min(jnp.where(ms == mx, iota_row, n)).astype(jnp.int32)
            sel = features_ref[b, pl.ds(idx, 1), :]
            out_ref[b, pl.ds(r, 1), :] = sel
            dots = _chunked_rowdot(features_ref, b, sel, n, d)
            snorm = jnp.maximum(jnp.sqrt(jnp.sum(sel * sel)), EPS)
            sim = dots / (denom_row * snorm)
            mask_row = mask_row * (1.0 - jnp.clip(sim, 0.0, 1.0))


def kernel(batch_size, features, fallback):
    del batch_size, fallback
    b, n, d = features.shape
    return pl.pallas_call(
        _greedy_body,
        grid=(b // BB,),
        in_specs=[pl.BlockSpec((BB, n, d), lambda i: (i, 0, 0))],
        out_specs=pl.BlockSpec((BB, N_SLOTS_K, d), lambda i: (i, 0, 0)),
        out_shape=jax.ShapeDtypeStruct((b, N_SLOTS_K, d), features.dtype),
    )(features)
